# Initial kernel scaffold; baseline (speedup 1.0000x reference)
import jax
import jax.numpy as jnp
from jax.experimental import pallas as pl


def kernel(gsp_y_osgb_fourier, gsp_x_osgb_fourier, hrvsatellite_solar_azimuth, gsp_id, emb_table):
    raise NotImplementedError("write your pallas kernel here")


# speedup vs baseline: 1.5631x; 1.5631x over previous
import functools

import jax
import jax.numpy as jnp
from jax import lax
from jax.experimental import pallas as pl
from jax.experimental.pallas import tpu as pltpu
from jax.experimental.pallas import tpu_sc as plsc

B = 16384
F = 32
V = 1000
D = 128
R = 4
QC = 3 * F + D

NC = 2
NS = 16
NW = NC * NS
RW = B // NW
CE = 64
C4 = CE * R
G = RW // CE

_mesh = plsc.VectorSubcoreMesh(core_axis_name="c", subcore_axis_name="s")


@functools.partial(
    pl.kernel,
    mesh=_mesh,
    out_type=jax.ShapeDtypeStruct((B * R, QC), jnp.float32),
    scratch_types=[
        pltpu.VMEM((CE,), jnp.int32),
        pltpu.VMEM((CE, F), jnp.float32),
        pltpu.VMEM((CE, F), jnp.float32),
        pltpu.VMEM((CE, D), jnp.float32),
        pltpu.VMEM((C4, QC), jnp.float32),
        pltpu.SemaphoreType.DMA,
    ],
)
def _gsp_query_sc(y_hbm, x_hbm, ids_hbm, table_hbm, out_hbm,
                  eidx_v, y_v, x_v, emb_v, q4, sem):
    wid = lax.axis_index("s") * NC + lax.axis_index("c")
    base = wid * RW

    ones16 = jnp.ones((16,), jnp.float32)
    for i in range(C4):
        q4[i, pl.ds(0, 16)] = ones16
        q4[i, pl.ds(16, 16)] = ones16

    def group(m, carry):
        ex0 = base + m * CE
        pltpu.sync_copy(ids_hbm.at[pl.ds(ex0, CE)], eidx_v)
        pltpu.sync_copy(y_hbm.at[pl.ds(ex0, CE)], y_v)
        pltpu.sync_copy(x_hbm.at[pl.ds(ex0, CE)], x_v)
        pltpu.async_copy(table_hbm.at[eidx_v], emb_v, sem).wait()
        for e in range(CE):
            for c in range(0, F, 16):
                yv = y_v[e, pl.ds(c, 16)]
                xv = x_v[e, pl.ds(c, 16)]
                for r in range(R):
                    q4[R * e + r, pl.ds(F + c, 16)] = yv
                    q4[R * e + r, pl.ds(2 * F + c, 16)] = xv
            for c in range(0, D, 16):
                ev = emb_v[e, pl.ds(c, 16)]
                for r in range(R):
                    q4[R * e + r, pl.ds(3 * F + c, 16)] = ev
        pltpu.sync_copy(q4, out_hbm.at[pl.ds(R * ex0 + 0, C4)])
        return carry

    lax.fori_loop(0, G, group, 0)


def kernel(gsp_y_osgb_fourier, gsp_x_osgb_fourier, hrvsatellite_solar_azimuth,
           gsp_id, emb_table):
    y = gsp_y_osgb_fourier[:, 0, :]
    x = gsp_x_osgb_fourier[:, 0, :]
    ids = gsp_id[:, 0]
    n_repeats = hrvsatellite_solar_azimuth.shape[0] // B
    assert n_repeats == R
    return _gsp_query_sc(y, x, ids, emb_table).reshape(B * R, 1, QC)
